# baseline (speedup 1.0000x reference)
import functools

import jax
import jax.numpy as jnp
from jax import lax
from jax.experimental import pallas as pl
from jax.experimental.pallas import tpu as pltpu
from jax.experimental.pallas import tpu_sc as plsc

ROWS = 1024
COLS = 100000
K = 200

NC, NS, L = 2, 16, 16
NW = NC * NS
TOTAL = ROWS * K
PER_W = TOTAL // NW
CHUNK = 128
NCHUNK = PER_W // CHUNK
NVEC = PER_W // L

_mesh = plsc.VectorSubcoreMesh(core_axis_name="c", subcore_axis_name="s")


@functools.partial(
    pl.kernel,
    mesh=_mesh,
    out_type=jax.ShapeDtypeStruct((TOTAL,), jnp.float32),
    scratch_types=[
        pltpu.VMEM((PER_W,), jnp.int32),
        pltpu.VMEM((PER_W,), jnp.int32),
        pltpu.VMEM((PER_W,), jnp.float32),
        pltpu.SemaphoreType.DMA,
    ],
)
def _sc_gather(x_hbm, idx_hbm, out_hbm, idx_v, gidx_v, vals_v, sem):
    wid = lax.axis_index("s") * NC + lax.axis_index("c")
    base = wid * PER_W

    pltpu.sync_copy(idx_hbm.at[pl.ds(base, PER_W)], idx_v)

    lanes = lax.broadcasted_iota(jnp.int32, (L,), 0)

    def to_flat(v, carry):
        row, rem = carry
        sl = pl.ds(v * L, L)
        row_vec = row + jnp.where(lanes >= K - rem, 1, 0)
        gidx_v[sl] = idx_v[sl] + row_vec * COLS
        rem = rem + L
        crossed = rem >= K
        return (jnp.where(crossed, row + 1, row), jnp.where(crossed, rem - K, rem))

    lax.fori_loop(0, NVEC, to_flat, (wid * (PER_W // K), jnp.int32(0)))

    @pl.loop(0, NCHUNK)
    def _(j):
        sl = pl.ds(j * CHUNK, CHUNK)
        pltpu.async_copy(x_hbm.at[gidx_v.at[sl]], vals_v.at[sl], sem)

    @pl.loop(0, NCHUNK)
    def _(j):
        sl = pl.ds(j * CHUNK, CHUNK)
        pltpu.make_async_copy(x_hbm.at[gidx_v.at[sl]], vals_v.at[sl], sem).wait()

    pltpu.sync_copy(vals_v, out_hbm.at[pl.ds(base, PER_W)])


def kernel(x, indices):
    out = _sc_gather(x.reshape(-1), indices.reshape(-1))
    return out.reshape(ROWS, K)


# speedup vs baseline: 2.3172x; 2.3172x over previous
import functools

import jax
import jax.numpy as jnp
from jax import lax
from jax.experimental import pallas as pl
from jax.experimental.pallas import tpu as pltpu
from jax.experimental.pallas import tpu_sc as plsc

ROWS = 1024
COLS = 100000
K = 200

LANES = 128
SUBL = 8
TILE_COLS = -(-COLS // LANES)

NC, NS, L = 2, 16, 16
NW = NC * NS
TOTAL = ROWS * K
PER_W = TOTAL // NW
CHUNK = 128
NCHUNK = PER_W // CHUNK
NVEC = PER_W // L

_mesh = plsc.VectorSubcoreMesh(core_axis_name="c", subcore_axis_name="s")


@functools.partial(
    pl.kernel,
    mesh=_mesh,
    out_type=jax.ShapeDtypeStruct((TOTAL,), jnp.float32),
    compiler_params=pltpu.CompilerParams(disable_bounds_checks=True),
    scratch_types=[
        pltpu.VMEM((PER_W,), jnp.int32),
        pltpu.VMEM((PER_W,), jnp.int32),
        pltpu.VMEM((PER_W,), jnp.float32),
        pltpu.SemaphoreType.DMA,
    ],
)
def _sc_gather(x_hbm, idx_hbm, out_hbm, idx_v, gidx_v, vals_v, sem):
    wid = lax.axis_index("s") * NC + lax.axis_index("c")
    base = wid * PER_W

    pltpu.sync_copy(idx_hbm.at[pl.ds(base, PER_W)], idx_v)

    lanes = lax.broadcasted_iota(jnp.int32, (L,), 0)

    def to_phys(v, carry):
        row, rem = carry
        sl = pl.ds(v * L, L)
        i = row + jnp.where(lanes >= K - rem, 1, 0)
        j = idx_v[sl]
        gidx_v[sl] = (
            (((i >> 3) * TILE_COLS + (j >> 7)) << 10)
            + ((i & (SUBL - 1)) << 7)
            + (j & (LANES - 1))
        )
        rem = rem + L
        crossed = rem >= K
        return (jnp.where(crossed, row + 1, row), jnp.where(crossed, rem - K, rem))

    lax.fori_loop(0, NVEC, to_phys, (wid * (PER_W // K), jnp.int32(0)))

    x_flat = x_hbm.at[0, pl.ds(0, CHUNK)]

    @pl.loop(0, NCHUNK)
    def _(j):
        sl = pl.ds(j * CHUNK, CHUNK)
        pltpu.async_copy(x_flat.at[gidx_v.at[sl]], vals_v.at[sl], sem)

    @pl.loop(0, NCHUNK)
    def _(j):
        sl = pl.ds(j * CHUNK, CHUNK)
        pltpu.make_async_copy(x_flat.at[gidx_v.at[sl]], vals_v.at[sl], sem).wait()

    pltpu.sync_copy(vals_v, out_hbm.at[pl.ds(base, PER_W)])


def kernel(x, indices):
    out = _sc_gather(x, indices.reshape(-1))
    return out.reshape(ROWS, K)


# speedup vs baseline: 27.6091x; 11.9146x over previous
import functools

import jax
import jax.numpy as jnp
from jax import lax
from jax.experimental import pallas as pl
from jax.experimental.pallas import tpu as pltpu
from jax.experimental.pallas import tpu_sc as plsc

ROWS = 1024
COLS = 100000
K = 200

NC, NS, L = 2, 16, 16
NW = NC * NS
TOTAL = ROWS * K
PER_W = TOTAL // NW
CHUNK = 128
NCHUNK = PER_W // CHUNK
NVEC = PER_W // L

_mesh = plsc.VectorSubcoreMesh(core_axis_name="c", subcore_axis_name="s")


@functools.partial(
    pl.kernel,
    mesh=_mesh,
    out_type=jax.ShapeDtypeStruct((TOTAL,), jnp.float32),
    compiler_params=pltpu.CompilerParams(disable_bounds_checks=True),
    scratch_types=[
        pltpu.VMEM((PER_W,), jnp.int32),
        pltpu.VMEM((PER_W,), jnp.int32),
        pltpu.VMEM((PER_W,), jnp.float32),
        pltpu.SemaphoreType.DMA,
    ],
)
def _sc_gather(xt_hbm, idx_hbm, out_hbm, idx_v, gidx_v, vals_v, sem):
    wid = lax.axis_index("s") * NC + lax.axis_index("c")
    base = wid * PER_W

    pltpu.sync_copy(idx_hbm.at[pl.ds(base, PER_W)], idx_v)

    lanes = lax.broadcasted_iota(jnp.int32, (L,), 0)

    @pl.loop(0, NVEC)
    def _(v):
        sl = pl.ds(v * L, L)
        i = (base + v * L + lanes) & (ROWS - 1)
        j = idx_v[sl]
        gidx_v[sl] = (
            ((j >> 3) << 13) + ((i >> 7) << 10) + ((j & 7) << 7) + (i & 127)
        )

    x_flat = xt_hbm.at[0, pl.ds(0, CHUNK)]

    @pl.loop(0, NCHUNK)
    def _(j):
        sl = pl.ds(j * CHUNK, CHUNK)
        pltpu.async_copy(x_flat.at[gidx_v.at[sl]], vals_v.at[sl], sem)

    @pl.loop(0, NCHUNK)
    def _(j):
        sl = pl.ds(j * CHUNK, CHUNK)
        pltpu.make_async_copy(x_flat.at[gidx_v.at[sl]], vals_v.at[sl], sem).wait()

    pltpu.sync_copy(vals_v, out_hbm.at[pl.ds(base, PER_W)])


def kernel(x, indices):
    out = _sc_gather(x.T, indices.T.reshape(-1))
    return out.reshape(K, ROWS).T


# speedup vs baseline: 28.7654x; 1.0419x over previous
import functools

import jax
import jax.numpy as jnp
from jax import lax
from jax.experimental import pallas as pl
from jax.experimental.pallas import tpu as pltpu
from jax.experimental.pallas import tpu_sc as plsc

ROWS = 1024
COLS = 100000
K = 200

NC, NS, L = 2, 16, 16
NW = NC * NS
TOTAL = ROWS * K
PER_W = TOTAL // NW
CHUNK = 128
NCHUNK = PER_W // CHUNK
NVEC = PER_W // L

_mesh = plsc.VectorSubcoreMesh(core_axis_name="c", subcore_axis_name="s")


@functools.partial(
    pl.kernel,
    mesh=_mesh,
    out_type=jax.ShapeDtypeStruct((TOTAL,), jnp.float32),
    compiler_params=pltpu.CompilerParams(disable_bounds_checks=True),
    scratch_types=[
        pltpu.VMEM((PER_W,), jnp.int32),
        pltpu.VMEM((PER_W,), jnp.int32),
        pltpu.VMEM((PER_W,), jnp.float32),
        pltpu.SemaphoreType.DMA,
    ],
)
def _sc_gather(xt_hbm, idx_hbm, out_hbm, idx_v, gidx_v, vals_v, sem):
    wid = lax.axis_index("s") * NC + lax.axis_index("c")
    base = wid * PER_W

    pltpu.sync_copy(idx_hbm.at[pl.ds(base, PER_W)], idx_v)

    lanes = lax.broadcasted_iota(jnp.int32, (L,), 0)

    x_flat = xt_hbm.at[0, pl.ds(0, CHUNK)]

    @pl.loop(0, NCHUNK)
    def _(jc):
        for c in range(CHUNK // L):
            sl = pl.ds(jc * CHUNK + c * L, L)
            i = (base + jc * CHUNK + c * L + lanes)
            i = i & (ROWS - 1)
            j = idx_v[sl]
            gidx_v[sl] = (
                ((j >> 3) << 13) + ((i >> 7) << 10) + ((j & 7) << 7) + (i & 127)
            )
        csl = pl.ds(jc * CHUNK, CHUNK)
        pltpu.async_copy(x_flat.at[gidx_v.at[csl]], vals_v.at[csl], sem)

    @pl.loop(0, NCHUNK)
    def _(j):
        sl = pl.ds(j * CHUNK, CHUNK)
        pltpu.make_async_copy(x_flat.at[gidx_v.at[sl]], vals_v.at[sl], sem).wait()

    pltpu.sync_copy(vals_v, out_hbm.at[pl.ds(base, PER_W)])


def kernel(x, indices):
    out = _sc_gather(x.T, indices.T.reshape(-1))
    return out.reshape(K, ROWS).T


# speedup vs baseline: 29.5526x; 1.0274x over previous
import functools

import jax
import jax.numpy as jnp
from jax import lax
from jax.experimental import pallas as pl
from jax.experimental.pallas import tpu as pltpu
from jax.experimental.pallas import tpu_sc as plsc

ROWS = 1024
COLS = 100000
K = 200

NC, NS, L = 2, 16, 16
NW = NC * NS
CHUNK = 128
NCG = 8
NKG = 4
K_START = (0, 56, 104, 152)
K_MAX = 56

_mesh = plsc.VectorSubcoreMesh(core_axis_name="c", subcore_axis_name="s")


@functools.partial(
    pl.kernel,
    mesh=_mesh,
    out_type=jax.ShapeDtypeStruct((K, ROWS), jnp.float32),
    compiler_params=pltpu.CompilerParams(disable_bounds_checks=True),
    scratch_types=[
        pltpu.VMEM((K_MAX, CHUNK), jnp.int32),
        pltpu.VMEM((K_MAX, CHUNK), jnp.int32),
        pltpu.VMEM((K_MAX, CHUNK), jnp.float32),
        pltpu.SemaphoreType.DMA,
    ],
)
def _sc_gather(xt_hbm, idx_hbm, out_hbm, idx_v, gidx_v, vals_v, sem):
    wid = lax.axis_index("s") * NC + lax.axis_index("c")
    cg = wid & (NCG - 1)
    kg = wid >> 3
    k0 = (kg > 0) * 8 + kg * 48
    klen = jnp.where(kg == 0, K_MAX, 48)

    pltpu.sync_copy(
        idx_hbm.at[pl.ds(k0, 48), pl.ds(cg * CHUNK, CHUNK)],
        idx_v.at[pl.ds(0, 48)],
    )

    @pl.when(kg == 0)
    def _():
        pltpu.sync_copy(
            idx_hbm.at[pl.ds(48, 8), pl.ds(cg * CHUNK, CHUNK)],
            idx_v.at[pl.ds(48, 8)],
        )

    lanes = lax.broadcasted_iota(jnp.int32, (L,), 0)

    x_flat = xt_hbm.at[0, pl.ds(0, CHUNK)]

    @pl.loop(0, klen)
    def _(r):
        for c in range(CHUNK // L):
            icontrib = cg * 1024 + c * L + lanes
            j = idx_v[r, pl.ds(c * L, L)]
            gidx_v[r, pl.ds(c * L, L)] = (
                ((j >> 3) << 13) + ((j & 7) << 7) + icontrib
            )
        pltpu.async_copy(x_flat.at[gidx_v.at[r]], vals_v.at[r], sem)

    @pl.loop(0, klen)
    def _(r):
        pltpu.make_async_copy(x_flat.at[gidx_v.at[r]], vals_v.at[r], sem).wait()

    pltpu.sync_copy(
        vals_v.at[pl.ds(0, 48)],
        out_hbm.at[pl.ds(k0, 48), pl.ds(cg * CHUNK, CHUNK)],
    )

    @pl.when(kg == 0)
    def _():
        pltpu.sync_copy(
            vals_v.at[pl.ds(48, 8)],
            out_hbm.at[pl.ds(48, 8), pl.ds(cg * CHUNK, CHUNK)],
        )


def kernel(x, indices):
    out = _sc_gather(x.T, indices.T)
    return out.T
